# baseline (speedup 1.0000x reference)
import functools

import jax
import jax.numpy as jnp
from jax import lax
from jax.experimental import pallas as pl
from jax.experimental.pallas import tpu as pltpu
from jax.experimental.pallas import tpu_sc as plsc

N = 10000
E = 320000
H = 128

NC = 2
NS = 16
NW = NC * NS

K = 128
EPW = 10240
NCHUNK = EPW // K
E_PAD = NW * EPW

ACC_ROWS = 10240
ZERO_ROWS_PER_SUB = ACC_ROWS // NS

R_BLK = 1000
NB = N // R_BLK


NBUF = 2


def _sc_agg_body(x_hbm, src_hbm, dst_hbm, zeros_hbm, out_hbm,
                 src_cv, dst_v, rows_v, isem, gsem, ssem, acc_sh):
    c = lax.axis_index("c")
    s = lax.axis_index("s")
    wid = s * NC + c

    pltpu.sync_copy(
        zeros_hbm.at[pl.ds(s * ZERO_ROWS_PER_SUB, ZERO_ROWS_PER_SUB)],
        acc_sh.at[pl.ds(s * ZERO_ROWS_PER_SUB, ZERO_ROWS_PER_SUB)])

    pltpu.sync_copy(dst_hbm.at[wid], dst_v)

    plsc.subcore_barrier()

    pltpu.sync_copy(src_hbm.at[wid, 0], src_cv.at[0])
    pltpu.async_copy(x_hbm.at[src_cv.at[0]], rows_v.at[0], gsem.at[0])
    pltpu.async_copy(src_hbm.at[wid, 1], src_cv.at[1], isem.at[1])

    def chunk(j, carry):
        b = lax.rem(j, NBUF)
        bn = lax.rem(j + 1, NBUF)

        pltpu.make_async_copy(
            x_hbm.at[src_cv.at[b]], rows_v.at[b], gsem.at[b]).wait()

        @pl.when(j >= 1)
        def _wait_prev_scatter():
            pltpu.make_async_copy(
                rows_v.at[bn], acc_sh.at[dst_v.at[j - 1]],
                ssem.at[bn]).wait()

        pltpu.async_copy(
            rows_v.at[b], acc_sh.at[dst_v.at[j]], ssem.at[b], add=True)

        @pl.when(j + 1 < NCHUNK)
        def _fire_next_gather():
            pltpu.make_async_copy(
                src_hbm.at[wid, j + 1], src_cv.at[bn], isem.at[bn]).wait()
            pltpu.async_copy(
                x_hbm.at[src_cv.at[bn]], rows_v.at[bn], gsem.at[bn])

            @pl.when(j + 2 < NCHUNK)
            def _fire_next_idx():
                pltpu.async_copy(
                    src_hbm.at[wid, j + 2], src_cv.at[b], isem.at[b])

        return carry

    lax.fori_loop(0, NCHUNK, chunk, 0, unroll=False)

    bl = (NCHUNK - 1) % NBUF
    pltpu.make_async_copy(
        rows_v.at[bl], acc_sh.at[dst_v.at[NCHUNK - 1]], ssem.at[bl]).wait()

    plsc.subcore_barrier()

    pltpu.sync_copy(
        acc_sh.at[pl.ds(s * ZERO_ROWS_PER_SUB, ZERO_ROWS_PER_SUB)],
        out_hbm.at[c, pl.ds(s * ZERO_ROWS_PER_SUB, ZERO_ROWS_PER_SUB)])


@functools.lru_cache(maxsize=1)
def _make_sc_agg():
    return functools.partial(
        pl.kernel,
        out_type=jax.ShapeDtypeStruct((NC, ACC_ROWS, H), jnp.float32),
        mesh=plsc.VectorSubcoreMesh(
            core_axis_name="c", subcore_axis_name="s",
            num_cores=NC, num_subcores=NS),
        scratch_types=[
            pltpu.VMEM((NBUF, K), jnp.int32),
            pltpu.VMEM((NCHUNK, K), jnp.int32),
            pltpu.VMEM((NBUF, K, H), jnp.float32),
            pltpu.SemaphoreType.DMA((NBUF,)),
            pltpu.SemaphoreType.DMA((NBUF,)),
            pltpu.SemaphoreType.DMA((NBUF,)),
            pltpu.VMEM_SHARED((ACC_ROWS, H), jnp.float32),
        ],
    )(_sc_agg_body)


def _sc_agg(x, src, dst, zeros):
    return _make_sc_agg()(x, src, dst, zeros)


def _mlp_body(x_ref, p_ref, w1_ref, b1_ref, w2_ref, b2_ref,
              h2_ref, stats_ref):
    h = x_ref[...] + p_ref[0] + p_ref[1]
    h = jnp.dot(h, w1_ref[...], preferred_element_type=jnp.float32)
    h = jnp.maximum(h + b1_ref[...], 0.0)
    h2 = jnp.dot(h, w2_ref[...], preferred_element_type=jnp.float32)
    h2 = h2 + b2_ref[...]
    h2_ref[...] = h2

    @pl.when(pl.program_id(0) == 0)
    def _init():
        stats_ref[...] = jnp.zeros_like(stats_ref)

    stats_ref[0, :] += jnp.sum(h2, axis=0)
    stats_ref[1, :] += jnp.sum(h2 * h2, axis=0)


def _norm_body(h2_ref, stats_ref, gamma_ref, beta_ref, out_ref):
    mean = stats_ref[0, :] / N
    var = stats_ref[1, :] / N - mean * mean
    scale = lax.rsqrt(var + 1e-5) * gamma_ref[...]
    out_ref[...] = (h2_ref[...] - mean) * scale + beta_ref[...]


def kernel(x, edge_index, W1, b1, W2, b2, gamma, beta):
    src = edge_index[0]
    dst = edge_index[1]
    pad = E_PAD - E
    src = jnp.concatenate([src, jnp.zeros((pad,), jnp.int32)])
    dst = jnp.concatenate([dst, jnp.full((pad,), N, jnp.int32)])
    src = src.reshape(NW, NCHUNK, K)
    dst = dst.reshape(NW, NCHUNK, K)
    zeros = jnp.zeros((ACC_ROWS, H), jnp.float32)

    parts = _sc_agg(x, src, dst, zeros)

    h2, stats = pl.pallas_call(
        _mlp_body,
        grid=(NB,),
        in_specs=[
            pl.BlockSpec((R_BLK, H), lambda i: (i, 0)),
            pl.BlockSpec((2, R_BLK, H), lambda i: (0, i, 0)),
            pl.BlockSpec((H, H), lambda i: (0, 0)),
            pl.BlockSpec((H,), lambda i: (0,)),
            pl.BlockSpec((H, H), lambda i: (0, 0)),
            pl.BlockSpec((H,), lambda i: (0,)),
        ],
        out_specs=[
            pl.BlockSpec((R_BLK, H), lambda i: (i, 0)),
            pl.BlockSpec((8, H), lambda i: (0, 0)),
        ],
        out_shape=[
            jax.ShapeDtypeStruct((N, H), jnp.float32),
            jax.ShapeDtypeStruct((8, H), jnp.float32),
        ],
    )(x, parts, W1, b1, W2, b2)

    out = pl.pallas_call(
        _norm_body,
        grid=(NB,),
        in_specs=[
            pl.BlockSpec((R_BLK, H), lambda i: (i, 0)),
            pl.BlockSpec((8, H), lambda i: (0, 0)),
            pl.BlockSpec((H,), lambda i: (0,)),
            pl.BlockSpec((H,), lambda i: (0,)),
        ],
        out_specs=pl.BlockSpec((R_BLK, H), lambda i: (i, 0)),
        out_shape=jax.ShapeDtypeStruct((N, H), jnp.float32),
    )(h2, stats, gamma, beta)

    return out


# speedup vs baseline: 1.0340x; 1.0340x over previous
import functools

import jax
import jax.numpy as jnp
from jax import lax
from jax.experimental import pallas as pl
from jax.experimental.pallas import tpu as pltpu
from jax.experimental.pallas import tpu_sc as plsc

N = 10000
E = 320000
H = 128

NC = 2
NS = 16
NW = NC * NS

K = 64
EPW = 10240
NCHUNK = EPW // K
E_PAD = NW * EPW

ACC_ROWS = 10240
ZERO_ROWS_PER_SUB = ACC_ROWS // NS

R_BLK = 1000
NB = N // R_BLK


NBUF = 3


def _sc_agg_body(x_hbm, src_hbm, dst_hbm, zeros_hbm, out_hbm,
                 src_v, dst_v, rows_v, gsem, ssem, acc_sh):
    c = lax.axis_index("c")
    s = lax.axis_index("s")
    wid = s * NC + c

    pltpu.sync_copy(
        zeros_hbm.at[pl.ds(s * ZERO_ROWS_PER_SUB, ZERO_ROWS_PER_SUB)],
        acc_sh.at[pl.ds(s * ZERO_ROWS_PER_SUB, ZERO_ROWS_PER_SUB)])

    pltpu.sync_copy(src_hbm.at[wid], src_v)
    pltpu.sync_copy(dst_hbm.at[wid], dst_v)

    plsc.subcore_barrier()

    for b in range(NBUF - 1):
        pltpu.async_copy(
            x_hbm.at[src_v.at[pl.ds(b * K, K)]], rows_v.at[b], gsem.at[b])

    def chunk(j, carry):
        b = lax.rem(j, NBUF)
        bn = lax.rem(j + NBUF - 1, NBUF)
        jn = j + NBUF - 1

        @pl.when(jn < NCHUNK)
        def _fire_next_gather():
            @pl.when(j >= 1)
            def _wait_prev_scatter():
                pltpu.make_async_copy(
                    rows_v.at[bn], acc_sh.at[dst_v.at[pl.ds((j - 1) * K, K)]],
                    ssem.at[bn]).wait()

            pltpu.async_copy(
                x_hbm.at[src_v.at[pl.ds(jn * K, K)]], rows_v.at[bn],
                gsem.at[bn])

        pltpu.make_async_copy(
            x_hbm.at[src_v.at[pl.ds(j * K, K)]], rows_v.at[b],
            gsem.at[b]).wait()
        pltpu.async_copy(
            rows_v.at[b], acc_sh.at[dst_v.at[pl.ds(j * K, K)]],
            ssem.at[b], add=True)

        return carry

    lax.fori_loop(0, NCHUNK, chunk, 0, unroll=False)

    for t in range(NBUF):
        j = NCHUNK - NBUF + t
        bl = j % NBUF
        pltpu.make_async_copy(
            rows_v.at[bl], acc_sh.at[dst_v.at[pl.ds(j * K, K)]],
            ssem.at[bl]).wait()

    plsc.subcore_barrier()

    pltpu.sync_copy(
        acc_sh.at[pl.ds(s * ZERO_ROWS_PER_SUB, ZERO_ROWS_PER_SUB)],
        out_hbm.at[c, pl.ds(s * ZERO_ROWS_PER_SUB, ZERO_ROWS_PER_SUB)])


@functools.lru_cache(maxsize=1)
def _make_sc_agg():
    return functools.partial(
        pl.kernel,
        out_type=jax.ShapeDtypeStruct((NC, ACC_ROWS, H), jnp.float32),
        mesh=plsc.VectorSubcoreMesh(
            core_axis_name="c", subcore_axis_name="s",
            num_cores=NC, num_subcores=NS),
        scratch_types=[
            pltpu.VMEM((EPW,), jnp.int32),
            pltpu.VMEM((EPW,), jnp.int32),
            pltpu.VMEM((NBUF, K, H), jnp.float32),
            pltpu.SemaphoreType.DMA((NBUF,)),
            pltpu.SemaphoreType.DMA((NBUF,)),
            pltpu.VMEM_SHARED((ACC_ROWS, H), jnp.float32),
        ],
    )(_sc_agg_body)


def _sc_agg(x, src, dst, zeros):
    return _make_sc_agg()(x, src, dst, zeros)


def _mlp_body(x_ref, p_ref, w1_ref, b1_ref, w2_ref, b2_ref,
              h2_ref, stats_ref):
    h = x_ref[...] + p_ref[0] + p_ref[1]
    h = jnp.dot(h, w1_ref[...], preferred_element_type=jnp.float32)
    h = jnp.maximum(h + b1_ref[...], 0.0)
    h2 = jnp.dot(h, w2_ref[...], preferred_element_type=jnp.float32)
    h2 = h2 + b2_ref[...]
    h2_ref[...] = h2

    @pl.when(pl.program_id(0) == 0)
    def _init():
        stats_ref[...] = jnp.zeros_like(stats_ref)

    stats_ref[0, :] += jnp.sum(h2, axis=0)
    stats_ref[1, :] += jnp.sum(h2 * h2, axis=0)


def _norm_body(h2_ref, stats_ref, gamma_ref, beta_ref, out_ref):
    mean = stats_ref[0, :] / N
    var = stats_ref[1, :] / N - mean * mean
    scale = lax.rsqrt(var + 1e-5) * gamma_ref[...]
    out_ref[...] = (h2_ref[...] - mean) * scale + beta_ref[...]


def kernel(x, edge_index, W1, b1, W2, b2, gamma, beta):
    src = edge_index[0]
    dst = edge_index[1]
    pad = E_PAD - E
    src = jnp.concatenate([src, jnp.zeros((pad,), jnp.int32)])
    dst = jnp.concatenate([dst, jnp.full((pad,), N, jnp.int32)])
    src = src.reshape(NW, EPW)
    dst = dst.reshape(NW, EPW)
    zeros = jnp.zeros((ACC_ROWS, H), jnp.float32)

    parts = _sc_agg(x, src, dst, zeros)

    h2, stats = pl.pallas_call(
        _mlp_body,
        grid=(NB,),
        in_specs=[
            pl.BlockSpec((R_BLK, H), lambda i: (i, 0)),
            pl.BlockSpec((2, R_BLK, H), lambda i: (0, i, 0)),
            pl.BlockSpec((H, H), lambda i: (0, 0)),
            pl.BlockSpec((H,), lambda i: (0,)),
            pl.BlockSpec((H, H), lambda i: (0, 0)),
            pl.BlockSpec((H,), lambda i: (0,)),
        ],
        out_specs=[
            pl.BlockSpec((R_BLK, H), lambda i: (i, 0)),
            pl.BlockSpec((8, H), lambda i: (0, 0)),
        ],
        out_shape=[
            jax.ShapeDtypeStruct((N, H), jnp.float32),
            jax.ShapeDtypeStruct((8, H), jnp.float32),
        ],
    )(x, parts, W1, b1, W2, b2)

    out = pl.pallas_call(
        _norm_body,
        grid=(NB,),
        in_specs=[
            pl.BlockSpec((R_BLK, H), lambda i: (i, 0)),
            pl.BlockSpec((8, H), lambda i: (0, 0)),
            pl.BlockSpec((H,), lambda i: (0,)),
            pl.BlockSpec((H,), lambda i: (0,)),
        ],
        out_specs=pl.BlockSpec((R_BLK, H), lambda i: (i, 0)),
        out_shape=jax.ShapeDtypeStruct((N, H), jnp.float32),
    )(h2, stats, gamma, beta)

    return out


# speedup vs baseline: 3.9391x; 3.8097x over previous
import functools

import jax
import jax.numpy as jnp
from jax import lax
from jax.experimental import pallas as pl
from jax.experimental.pallas import tpu as pltpu
from jax.experimental.pallas import tpu_sc as plsc

N = 10000
E = 320000
H = 128

NC = 2
NS = 16
NW = NC * NS

K = 40
EPW = E // NW
NCHUNK = EPW // K

ACC_ROWS = 10240
ZERO_ROWS_PER_SUB = ACC_ROWS // NS

R_BLK = 1000
NB = N // R_BLK


NBUF = 4


def _sc_agg_body(x_hbm, src_hbm, dst_hbm, zeros_hbm, out_hbm,
                 src_v, dst_v, rows_v, gsem, ssem, acc_sh):
    c = lax.axis_index("c")
    s = lax.axis_index("s")
    wid = s * NC + c

    pltpu.sync_copy(
        zeros_hbm.at[pl.ds(s * ZERO_ROWS_PER_SUB, ZERO_ROWS_PER_SUB)],
        acc_sh.at[pl.ds(s * ZERO_ROWS_PER_SUB, ZERO_ROWS_PER_SUB)])

    pltpu.sync_copy(src_hbm.at[pl.ds(wid * EPW, EPW)], src_v)
    pltpu.sync_copy(dst_hbm.at[pl.ds(wid * EPW, EPW)], dst_v)

    plsc.subcore_barrier()

    for b in range(NBUF - 1):
        pltpu.async_copy(
            x_hbm.at[src_v.at[pl.ds(b * K, K)]], rows_v.at[b], gsem.at[b])

    def chunk(j, carry):
        b = lax.rem(j, NBUF)
        bn = lax.rem(j + NBUF - 1, NBUF)
        jn = j + NBUF - 1

        @pl.when(jn < NCHUNK)
        def _fire_next_gather():
            @pl.when(j >= 1)
            def _wait_prev_scatter():
                pltpu.make_async_copy(
                    rows_v.at[bn], acc_sh.at[dst_v.at[pl.ds((j - 1) * K, K)]],
                    ssem.at[bn]).wait()

            pltpu.async_copy(
                x_hbm.at[src_v.at[pl.ds(jn * K, K)]], rows_v.at[bn],
                gsem.at[bn])

        pltpu.make_async_copy(
            x_hbm.at[src_v.at[pl.ds(j * K, K)]], rows_v.at[b],
            gsem.at[b]).wait()
        pltpu.async_copy(
            rows_v.at[b], acc_sh.at[dst_v.at[pl.ds(j * K, K)]],
            ssem.at[b], add=True)

        return carry

    lax.fori_loop(0, NCHUNK, chunk, 0, unroll=False)

    for t in range(NBUF):
        j = NCHUNK - NBUF + t
        bl = j % NBUF
        pltpu.make_async_copy(
            rows_v.at[bl], acc_sh.at[dst_v.at[pl.ds(j * K, K)]],
            ssem.at[bl]).wait()

    plsc.subcore_barrier()

    pltpu.sync_copy(
        acc_sh.at[pl.ds(s * ZERO_ROWS_PER_SUB, ZERO_ROWS_PER_SUB)],
        out_hbm.at[c, pl.ds(s * ZERO_ROWS_PER_SUB, ZERO_ROWS_PER_SUB)])


@functools.lru_cache(maxsize=1)
def _make_sc_agg():
    return functools.partial(
        pl.kernel,
        out_type=jax.ShapeDtypeStruct((NC, ACC_ROWS, H), jnp.float32),
        mesh=plsc.VectorSubcoreMesh(
            core_axis_name="c", subcore_axis_name="s",
            num_cores=NC, num_subcores=NS),
        scratch_types=[
            pltpu.VMEM((EPW,), jnp.int32),
            pltpu.VMEM((EPW,), jnp.int32),
            pltpu.VMEM((NBUF, K, H), jnp.float32),
            pltpu.SemaphoreType.DMA((NBUF,)),
            pltpu.SemaphoreType.DMA((NBUF,)),
            pltpu.VMEM_SHARED((ACC_ROWS, H), jnp.float32),
        ],
    )(_sc_agg_body)


def _sc_agg(x, src, dst, zeros):
    return _make_sc_agg()(x, src, dst, zeros)


def _mlp_body(x_ref, p_ref, w1_ref, b1_ref, w2_ref, b2_ref,
              h2_ref, stats_ref):
    h = x_ref[...] + p_ref[0] + p_ref[1]
    h = jnp.dot(h, w1_ref[...], preferred_element_type=jnp.float32)
    h = jnp.maximum(h + b1_ref[...], 0.0)
    h2 = jnp.dot(h, w2_ref[...], preferred_element_type=jnp.float32)
    h2 = h2 + b2_ref[...]
    h2_ref[...] = h2

    @pl.when(pl.program_id(0) == 0)
    def _init():
        stats_ref[...] = jnp.zeros_like(stats_ref)

    stats_ref[0, :] += jnp.sum(h2, axis=0)
    stats_ref[1, :] += jnp.sum(h2 * h2, axis=0)


def _norm_body(h2_ref, stats_ref, gamma_ref, beta_ref, out_ref):
    mean = stats_ref[0, :] / N
    var = stats_ref[1, :] / N - mean * mean
    scale = lax.rsqrt(var + 1e-5) * gamma_ref[...]
    out_ref[...] = (h2_ref[...] - mean) * scale + beta_ref[...]


def kernel(x, edge_index, W1, b1, W2, b2, gamma, beta):
    src = edge_index[0]
    dst = edge_index[1]
    zeros = jnp.zeros((ACC_ROWS, H), jnp.float32)

    parts = _sc_agg(x, src, dst, zeros)

    h2, stats = pl.pallas_call(
        _mlp_body,
        grid=(NB,),
        in_specs=[
            pl.BlockSpec((R_BLK, H), lambda i: (i, 0)),
            pl.BlockSpec((2, R_BLK, H), lambda i: (0, i, 0)),
            pl.BlockSpec((H, H), lambda i: (0, 0)),
            pl.BlockSpec((H,), lambda i: (0,)),
            pl.BlockSpec((H, H), lambda i: (0, 0)),
            pl.BlockSpec((H,), lambda i: (0,)),
        ],
        out_specs=[
            pl.BlockSpec((R_BLK, H), lambda i: (i, 0)),
            pl.BlockSpec((8, H), lambda i: (0, 0)),
        ],
        out_shape=[
            jax.ShapeDtypeStruct((N, H), jnp.float32),
            jax.ShapeDtypeStruct((8, H), jnp.float32),
        ],
    )(x, parts, W1, b1, W2, b2)

    out = pl.pallas_call(
        _norm_body,
        grid=(NB,),
        in_specs=[
            pl.BlockSpec((R_BLK, H), lambda i: (i, 0)),
            pl.BlockSpec((8, H), lambda i: (0, 0)),
            pl.BlockSpec((H,), lambda i: (0,)),
            pl.BlockSpec((H,), lambda i: (0,)),
        ],
        out_specs=pl.BlockSpec((R_BLK, H), lambda i: (i, 0)),
        out_shape=jax.ShapeDtypeStruct((N, H), jnp.float32),
    )(h2, stats, gamma, beta)

    return out


# speedup vs baseline: 4.1004x; 1.0409x over previous
import functools

import jax
import jax.numpy as jnp
from jax import lax
from jax.experimental import pallas as pl
from jax.experimental.pallas import tpu as pltpu
from jax.experimental.pallas import tpu_sc as plsc

N = 10000
E = 320000
H = 128

NC = 2
NS = 16
NW = NC * NS

K = 40
EPW = E // NW
NCHUNK = EPW // K

ACC_ROWS = 10240
ZERO_ROWS_PER_SUB = ACC_ROWS // NS

R_BLK = 1000
NB = N // R_BLK


NBUF = 5


def _sc_agg_body(x_hbm, src_hbm, dst_hbm, zeros_hbm, out_hbm,
                 src_v, dst_v, rows_v, gsem, ssem, acc_sh):
    c = lax.axis_index("c")
    s = lax.axis_index("s")
    wid = s * NC + c

    pltpu.sync_copy(
        zeros_hbm.at[pl.ds(s * ZERO_ROWS_PER_SUB, ZERO_ROWS_PER_SUB)],
        acc_sh.at[pl.ds(s * ZERO_ROWS_PER_SUB, ZERO_ROWS_PER_SUB)])

    pltpu.sync_copy(src_hbm.at[pl.ds(wid * EPW, EPW)], src_v)
    pltpu.sync_copy(dst_hbm.at[pl.ds(wid * EPW, EPW)], dst_v)

    plsc.subcore_barrier()

    for b in range(NBUF - 1):
        pltpu.async_copy(
            x_hbm.at[src_v.at[pl.ds(b * K, K)]], rows_v.at[b], gsem.at[b])

    def chunk(j, carry):
        b = lax.rem(j, NBUF)
        bn = lax.rem(j + NBUF - 1, NBUF)
        jn = j + NBUF - 1

        @pl.when(jn < NCHUNK)
        def _fire_next_gather():
            @pl.when(j >= 1)
            def _wait_prev_scatter():
                pltpu.make_async_copy(
                    rows_v.at[bn], acc_sh.at[dst_v.at[pl.ds((j - 1) * K, K)]],
                    ssem.at[bn]).wait()

            pltpu.async_copy(
                x_hbm.at[src_v.at[pl.ds(jn * K, K)]], rows_v.at[bn],
                gsem.at[bn])

        pltpu.make_async_copy(
            x_hbm.at[src_v.at[pl.ds(j * K, K)]], rows_v.at[b],
            gsem.at[b]).wait()
        pltpu.async_copy(
            rows_v.at[b], acc_sh.at[dst_v.at[pl.ds(j * K, K)]],
            ssem.at[b], add=True)

        return carry

    lax.fori_loop(0, NCHUNK, chunk, 0, unroll=False)

    for t in range(NBUF):
        j = NCHUNK - NBUF + t
        bl = j % NBUF
        pltpu.make_async_copy(
            rows_v.at[bl], acc_sh.at[dst_v.at[pl.ds(j * K, K)]],
            ssem.at[bl]).wait()

    plsc.subcore_barrier()

    pltpu.sync_copy(
        acc_sh.at[pl.ds(s * ZERO_ROWS_PER_SUB, ZERO_ROWS_PER_SUB)],
        out_hbm.at[c, pl.ds(s * ZERO_ROWS_PER_SUB, ZERO_ROWS_PER_SUB)])


@functools.lru_cache(maxsize=1)
def _make_sc_agg():
    return functools.partial(
        pl.kernel,
        out_type=jax.ShapeDtypeStruct((NC, ACC_ROWS, H), jnp.float32),
        mesh=plsc.VectorSubcoreMesh(
            core_axis_name="c", subcore_axis_name="s",
            num_cores=NC, num_subcores=NS),
        scratch_types=[
            pltpu.VMEM((EPW,), jnp.int32),
            pltpu.VMEM((EPW,), jnp.int32),
            pltpu.VMEM((NBUF, K, H), jnp.float32),
            pltpu.SemaphoreType.DMA((NBUF,)),
            pltpu.SemaphoreType.DMA((NBUF,)),
            pltpu.VMEM_SHARED((ACC_ROWS, H), jnp.float32),
        ],
    )(_sc_agg_body)


def _sc_agg(x, src, dst, zeros):
    return _make_sc_agg()(x, src, dst, zeros)


def _mlp_body(x_ref, p_ref, w1_ref, b1_ref, w2_ref, b2_ref,
              h2_ref, stats_ref):
    h = x_ref[...] + p_ref[0] + p_ref[1]
    h = jnp.dot(h, w1_ref[...], preferred_element_type=jnp.float32)
    h = jnp.maximum(h + b1_ref[...], 0.0)
    h2 = jnp.dot(h, w2_ref[...], preferred_element_type=jnp.float32)
    h2 = h2 + b2_ref[...]
    h2_ref[...] = h2

    @pl.when(pl.program_id(0) == 0)
    def _init():
        stats_ref[...] = jnp.zeros_like(stats_ref)

    stats_ref[0, :] += jnp.sum(h2, axis=0)
    stats_ref[1, :] += jnp.sum(h2 * h2, axis=0)


def _norm_body(h2_ref, stats_ref, gamma_ref, beta_ref, out_ref):
    mean = stats_ref[0, :] / N
    var = stats_ref[1, :] / N - mean * mean
    scale = lax.rsqrt(var + 1e-5) * gamma_ref[...]
    out_ref[...] = (h2_ref[...] - mean) * scale + beta_ref[...]


def kernel(x, edge_index, W1, b1, W2, b2, gamma, beta):
    src = edge_index[0]
    dst = edge_index[1]
    zeros = jnp.zeros((ACC_ROWS, H), jnp.float32)

    parts = _sc_agg(x, src, dst, zeros)

    h2, stats = pl.pallas_call(
        _mlp_body,
        grid=(NB,),
        in_specs=[
            pl.BlockSpec((R_BLK, H), lambda i: (i, 0)),
            pl.BlockSpec((2, R_BLK, H), lambda i: (0, i, 0)),
            pl.BlockSpec((H, H), lambda i: (0, 0)),
            pl.BlockSpec((H,), lambda i: (0,)),
            pl.BlockSpec((H, H), lambda i: (0, 0)),
            pl.BlockSpec((H,), lambda i: (0,)),
        ],
        out_specs=[
            pl.BlockSpec((R_BLK, H), lambda i: (i, 0)),
            pl.BlockSpec((8, H), lambda i: (0, 0)),
        ],
        out_shape=[
            jax.ShapeDtypeStruct((N, H), jnp.float32),
            jax.ShapeDtypeStruct((8, H), jnp.float32),
        ],
    )(x, parts, W1, b1, W2, b2)

    out = pl.pallas_call(
        _norm_body,
        grid=(NB,),
        in_specs=[
            pl.BlockSpec((R_BLK, H), lambda i: (i, 0)),
            pl.BlockSpec((8, H), lambda i: (0, 0)),
            pl.BlockSpec((H,), lambda i: (0,)),
            pl.BlockSpec((H,), lambda i: (0,)),
        ],
        out_specs=pl.BlockSpec((R_BLK, H), lambda i: (i, 0)),
        out_shape=jax.ShapeDtypeStruct((N, H), jnp.float32),
    )(h2, stats, gamma, beta)

    return out
